# baseline (speedup 1.0000x reference)
import functools

import jax
import jax.numpy as jnp
from jax import lax
from jax.experimental import pallas as pl
from jax.experimental.pallas import tpu as pltpu
from jax.experimental.pallas import tpu_sc as plsc


_CHUNKS = (56, 56, 16)


def _make_sc_copy(b, s, d, dtype):
    info = plsc.get_sparse_core_info()
    nw = info.num_cores * info.num_subcores
    rows_per_w = s // nw
    chunks = _CHUNKS
    assert sum(chunks) == rows_per_w
    nbuf = 2
    bufrows = max(chunks)
    n_ch = len(chunks)
    offs = [sum(chunks[:i]) for i in range(n_ch)]
    mesh = plsc.VectorSubcoreMesh(core_axis_name="c", subcore_axis_name="s")

    @functools.partial(
        pl.kernel,
        mesh=mesh,
        out_type=jax.ShapeDtypeStruct((b * s, d), dtype),
        scratch_types=[
            pltpu.VMEM((nbuf * bufrows, d), dtype),
            pltpu.SemaphoreType.DMA,
            pltpu.SemaphoreType.DMA,
        ],
    )
    def sc_copy(emb_hbm, out_hbm, *scratch):
        allbuf, rsem, wsem = scratch
        bufs = [allbuf.at[pl.ds(i * bufrows, bufrows)] for i in range(nbuf)]
        rsems = [rsem] * nbuf
        wsems = [wsem] * nbuf
        wid = lax.axis_index("s") * info.num_cores + lax.axis_index("c")
        base = wid * rows_per_w
        reads = [None] * nbuf
        writes = [[] for _ in range(nbuf)]

        def start_read(c):
            reads[c % nbuf] = pltpu.async_copy(
                emb_hbm.at[pl.ds(base + offs[c], chunks[c])],
                bufs[c % nbuf].at[pl.ds(0, chunks[c])],
                rsems[c % nbuf])

        for c in range(min(nbuf, n_ch)):
            start_read(c)
        for c in range(n_ch):
            k = c % nbuf
            reads[k].wait()
            row0 = base + offs[c]
            for bb in range(b):
                writes[k].append(
                    pltpu.async_copy(bufs[k].at[pl.ds(0, chunks[c])],
                                     out_hbm.at[pl.ds(bb * s + row0,
                                                      chunks[c])],
                                     wsems[k]))
            r = c + 1
            if nbuf <= r < n_ch:
                kk = r % nbuf
                for wcp in writes[kk]:
                    wcp.wait()
                writes[kk] = []
                start_read(r)
        for side in writes:
            for wcp in side:
                wcp.wait()

    return sc_copy


def kernel(x, emb):
    b, s = x.shape
    max_seq_len, d = emb.shape
    assert s < max_seq_len
    out = _make_sc_copy(b, s, d, emb.dtype)(emb)
    return out.reshape(b, s, d)


# speedup vs baseline: 1.0073x; 1.0073x over previous
import functools

import jax
import jax.numpy as jnp
from jax import lax
from jax.experimental import pallas as pl
from jax.experimental.pallas import tpu as pltpu
from jax.experimental.pallas import tpu_sc as plsc


_CHUNKS = (56, 56, 16)


def _make_sc_copy(b, s, d, dtype):
    info = plsc.get_sparse_core_info()
    nw = info.num_cores * info.num_subcores
    rows_per_w = s // nw
    chunks = _CHUNKS
    assert sum(chunks) == rows_per_w
    nbuf = 2
    bufrows = max(chunks)
    n_ch = len(chunks)
    offs = [sum(chunks[:i]) for i in range(n_ch)]
    mesh = plsc.VectorSubcoreMesh(core_axis_name="c", subcore_axis_name="s")

    @functools.partial(
        pl.kernel,
        mesh=mesh,
        out_type=jax.ShapeDtypeStruct((b * s, d), dtype),
    )
    def sc_copy(emb_hbm, out_hbm):
        pl.run_scoped(
            functools.partial(_body, emb_hbm, out_hbm),
            pltpu.VMEM((nbuf * bufrows, d), dtype),
            pltpu.SemaphoreType.DMA,
            pltpu.SemaphoreType.DMA,
        )

    def _body(emb_hbm, out_hbm, allbuf, rsem, wsem):
        bufs = [allbuf.at[pl.ds(i * bufrows, bufrows)] for i in range(nbuf)]
        rsems = [rsem] * nbuf
        wsems = [wsem] * nbuf
        wid = lax.axis_index("s") * info.num_cores + lax.axis_index("c")
        base = wid * rows_per_w
        reads = [None] * nbuf
        writes = [[] for _ in range(nbuf)]

        def start_read(c):
            reads[c % nbuf] = pltpu.async_copy(
                emb_hbm.at[pl.ds(base + offs[c], chunks[c])],
                bufs[c % nbuf].at[pl.ds(0, chunks[c])],
                rsems[c % nbuf])

        for c in range(min(nbuf, n_ch)):
            start_read(c)
        for c in range(n_ch):
            k = c % nbuf
            reads[k].wait()
            row0 = base + offs[c]
            for bb in range(b):
                writes[k].append(
                    pltpu.async_copy(bufs[k].at[pl.ds(0, chunks[c])],
                                     out_hbm.at[pl.ds(bb * s + row0,
                                                      chunks[c])],
                                     wsems[k]))
            r = c + 1
            if nbuf <= r < n_ch:
                kk = r % nbuf
                for wcp in writes[kk]:
                    wcp.wait()
                writes[kk] = []
                start_read(r)
        for side in writes:
            for wcp in side:
                wcp.wait()

    return sc_copy


def kernel(x, emb):
    b, s = x.shape
    max_seq_len, d = emb.shape
    assert s < max_seq_len
    out = _make_sc_copy(b, s, d, emb.dtype)(emb)
    return out.reshape(b, s, d)


# speedup vs baseline: 1.0076x; 1.0003x over previous
import functools

import jax
from jax import lax
from jax.experimental import pallas as pl
from jax.experimental.pallas import tpu as pltpu
from jax.experimental.pallas import tpu_sc as plsc


_CHUNKS = (56, 56, 16)


def _make_sc_copy(b, s, d, dtype):
    info = plsc.get_sparse_core_info()
    nw = info.num_cores * info.num_subcores
    rows_per_w = s // nw
    chunks = _CHUNKS
    assert sum(chunks) == rows_per_w
    nbuf = 2
    bufrows = max(chunks)
    n_ch = len(chunks)
    offs = [sum(chunks[:i]) for i in range(n_ch)]
    mesh = plsc.VectorSubcoreMesh(core_axis_name="c", subcore_axis_name="s")

    @functools.partial(
        pl.kernel,
        mesh=mesh,
        out_type=jax.ShapeDtypeStruct((b * s, d), dtype),
    )
    def sc_copy(emb_hbm, out_hbm):
        pl.run_scoped(
            functools.partial(_body, emb_hbm, out_hbm),
            pltpu.VMEM((nbuf * bufrows, d), dtype),
            pltpu.SemaphoreType.DMA,
            pltpu.SemaphoreType.DMA,
        )

    def _body(emb_hbm, out_hbm, allbuf, rsem, wsem):
        bufs = [allbuf.at[pl.ds(i * bufrows, bufrows)] for i in range(nbuf)]
        rsems = [rsem] * nbuf
        wsems = [wsem] * nbuf
        wid = lax.axis_index("s") * info.num_cores + lax.axis_index("c")
        base = wid * rows_per_w
        reads = [None] * nbuf
        writes = [[] for _ in range(nbuf)]

        def start_read(c):
            reads[c % nbuf] = pltpu.async_copy(
                emb_hbm.at[pl.ds(base + offs[c], chunks[c])],
                bufs[c % nbuf].at[pl.ds(0, chunks[c])],
                rsems[c % nbuf])

        for c in range(min(nbuf, n_ch)):
            start_read(c)
        for c in range(n_ch):
            k = c % nbuf
            reads[k].wait()
            row0 = base + offs[c]
            for bb in range(b):
                writes[k].append(
                    pltpu.async_copy(bufs[k].at[pl.ds(0, chunks[c])],
                                     out_hbm.at[pl.ds(bb * s + row0,
                                                      chunks[c])],
                                     wsems[k]))
            r = c + 1
            if nbuf <= r < n_ch:
                kk = r % nbuf
                for wcp in writes[kk]:
                    wcp.wait()
                writes[kk] = []
                start_read(r)
        for side in writes:
            for wcp in side:
                wcp.wait()

    return sc_copy


def kernel(x, emb):
    b, s = x.shape
    max_seq_len, d = emb.shape
    assert s < max_seq_len
    out = _make_sc_copy(b, s, d, emb.dtype)(emb)
    return out.reshape(b, s, d)
